# baseline (speedup 1.0000x reference)
import functools

import jax
import jax.numpy as jnp
from jax.experimental import pallas as pl

MAXK = 100
NCAND = 128
NUM_CLASSES = 80
NUM_QUERY = 900


def _epilogue_body(s_r_ref, i_r_ref, s_c_ref, i_c_ref, bbox_ref, mult_ref,
                   scale_ref, out_ref, lab_ref):
    s_r = s_r_ref[...]
    i_r = i_r_ref[...]
    s_c = s_c_ref[...]
    i_c = i_c_ref[...]

    beats = (s_c > s_r) | ((s_c == s_r) & (i_c < i_r))
    rank = jnp.sum(beats.astype(jnp.int32), axis=0, keepdims=True)

    row_iota = jax.lax.broadcasted_iota(jnp.int32, (NCAND, NCAND), 0)
    P = row_iota == rank
    sorted_s = jnp.sum(jnp.where(P, s_r, 0.0), axis=1, keepdims=True)
    sorted_i = jnp.sum(jnp.where(P, i_r, 0), axis=1, keepdims=True)

    top_s = sorted_s[:MAXK, :]
    top_i = sorted_i[:MAXK, :]
    bbox_idx = top_i // NUM_CLASSES
    label = top_i % NUM_CLASSES

    col_iota = jax.lax.broadcasted_iota(jnp.int32, (MAXK, NUM_QUERY), 1)
    G = (bbox_idx == col_iota).astype(jnp.float32)
    cxcywh = jax.lax.dot_general(
        G, bbox_ref[...], (((1,), (0,)), ((), ())),
        precision=jax.lax.Precision.HIGHEST)

    cxcy = cxcywh[:, 0:2]
    wh = cxcywh[:, 2:4]
    x1y1 = cxcy - wh / 2
    x2y2 = cxcy + wh / 2
    boxes = jnp.concatenate([x1y1, x2y2], axis=1)
    mult = mult_ref[...]
    boxes = boxes * mult
    boxes = jnp.clip(boxes, jnp.zeros_like(mult), mult)
    boxes = boxes / scale_ref[...]
    out_ref[...] = jnp.concatenate([boxes, top_s], axis=1)
    lab_ref[...] = label


@functools.partial(jax.jit, static_argnames=())
def _epilogue(s128, i128, bbox, mult, scale):
    return pl.pallas_call(
        _epilogue_body,
        out_shape=[
            jax.ShapeDtypeStruct((MAXK, 5), jnp.float32),
            jax.ShapeDtypeStruct((MAXK, 1), jnp.int32),
        ],
    )(s128.reshape(1, NCAND), i128.reshape(1, NCAND),
      s128.reshape(NCAND, 1), i128.reshape(NCAND, 1),
      bbox, mult.reshape(1, 4), scale.reshape(1, 4))


def kernel(cls_logits, bbox_preds, scale, h, w):
    flat_scores = jax.nn.sigmoid(cls_logits[0]).reshape(-1)
    vals, idx = jax.lax.top_k(flat_scores, NCAND)
    s128 = vals
    i128 = idx.astype(jnp.int32)

    wf = jnp.float32(w)
    hf = jnp.float32(h)
    mult = (jnp.array([1.0, 0.0, 1.0, 0.0], jnp.float32) * wf
            + jnp.array([0.0, 1.0, 0.0, 1.0], jnp.float32) * hf)

    bboxes, labels = _epilogue(s128, i128, bbox_preds[0], mult, scale)
    return bboxes, labels.reshape(MAXK)


# speedup vs baseline: 2.9627x; 2.9627x over previous
import functools

import jax
import jax.numpy as jnp
from jax import lax
from jax.experimental import pallas as pl
from jax.experimental.pallas import tpu as pltpu
from jax.experimental.pallas import tpu_sc as plsc

MAXK = 100
NCAND = 128
NUM_CLASSES = 80
NUM_QUERY = 900

N = NUM_QUERY * NUM_CLASSES
NT = 16
CHUNK = 4512
NV = CHUNK // 16
NPAD = NT * CHUNK
NOUT = 144
MIN32 = -(2 ** 31)
M31 = 0x7FFFFFFF


def _bkey_from_f32(x):
    b = lax.bitcast_convert_type(x, jnp.int32)
    mask = ((lax.shift_right_arithmetic(b, 31) & M31) | MIN32)
    return b ^ mask


def _f32_from_bkey(bk):
    sk = bk ^ MIN32
    i = jnp.where(sk >= 0, sk, sk ^ M31)
    return lax.bitcast_convert_type(i, jnp.float32)


def _select_body(x_hbm, vals_hbm, idx_hbm,
                 chunk_v, keys_v, hist_v, cntbuf_v, merge_v,
                 ckeys_v, cidx_v, gtv_v, gti_v, eqi_v, cntm_v, tf_v,
                 shist_s, scnt_s, soutv_s, souti_s):
    tile = lax.axis_index("s")
    cid = lax.axis_index("c")
    base = tile * CHUNK
    lane = jnp.arange(16, dtype=jnp.int32)
    lane_base = lane * 256
    ones16 = jnp.ones((16,), jnp.int32)
    zeros16 = jnp.zeros((16,), jnp.int32)
    true16 = jnp.ones((16,), jnp.bool_)

    pltpu.sync_copy(x_hbm.at[pl.ds(base, CHUNK)], chunk_v)

    for w in range(256):
        hist_v[pl.ds(w * 16, 16)] = zeros16

    def r0_body(i, _):
        x = chunk_v[pl.ds(i * 16, 16)]
        bk = _bkey_from_f32(x)
        keys_v[pl.ds(i * 16, 16)] = bk
        digit = lax.shift_right_logical(bk, 24)
        plsc.addupdate_scatter(hist_v, [digit + lane_base], ones16,
                               mask=true16)
        return 0

    lax.fori_loop(0, NV, r0_body, 0)

    def reduce_and_publish():
        for v in range(16):
            acc = zeros16
            for l in range(16):
                acc = acc + hist_v[pl.ds(l * 256 + v * 16, 16)]
            cntbuf_v[pl.ds(v * 16, 16)] = acc
        pltpu.sync_copy(cntbuf_v, shist_s.at[pl.ds(tile * 256, 256)])
        plsc.subcore_barrier()
        pltpu.sync_copy(shist_s, merge_v)
        plsc.subcore_barrier()

    def walk(R):
        blocks = []
        tots = []
        for v in range(16):
            c = merge_v[pl.ds(v * 16, 16)]
            for t in range(1, 16):
                c = c + merge_v[pl.ds(t * 256 + v * 16, 16)]
            blocks.append(c)
            tots.append(jnp.sum(c))
        suffix = [jnp.int32(0)] * 17
        for v in range(15, -1, -1):
            suffix[v] = suffix[v + 1] + tots[v]
        dstar = jnp.int32(0)
        gstar = jnp.int32(0)
        done = jnp.int32(0)
        for v in range(16):
            c = blocks[v]
            sfx = lax.rev(plsc.cumsum(lax.rev(c, (0,))), (0,)) - c
            gs = suffix[v + 1] + sfx
            m = gs < R
            mi = m.astype(jnp.int32)
            found = jnp.max(mi)
            first = jnp.sum(jnp.int32(1) - mi)
            gfirst = jnp.max(jnp.where(m, gs, jnp.int32(-1)))
            sel = found * (jnp.int32(1) - done)
            dstar = dstar + sel * (v * 16 + first)
            gstar = gstar + sel * gfirst
            done = jnp.maximum(done, found)
        return dstar, R - gstar

    reduce_and_publish()
    R = jnp.int32(NCAND)
    d1, R = walk(R)
    prefix = d1

    def compact_body(i, nact):
        bk = keys_v[pl.ds(i * 16, 16)]
        m = lax.shift_right_logical(bk, 24) >= d1
        mi = m.astype(jnp.int32)
        pos = nact + plsc.cumsum(mi) - mi
        idxv = base + i * 16 + lane
        plsc.store_scatter(ckeys_v, [pos], bk, mask=m)
        plsc.store_scatter(cidx_v, [pos], idxv, mask=m)
        return nact + jnp.sum(mi)

    nact = lax.fori_loop(0, NV, compact_body, jnp.int32(0))
    nv_act = (nact + 15) // 16

    for rnd in range(1, 4):
        shift = 24 - 8 * rnd
        for w in range(256):
            hist_v[pl.ds(w * 16, 16)] = zeros16

        def r_body(i, _, shift=shift, prefix=prefix, nact=nact):
            bk = ckeys_v[pl.ds(i * 16, 16)]
            valid = (i * 16 + lane) < nact
            active = valid & (lax.shift_right_logical(bk, shift + 8) == prefix)
            digit = lax.shift_right_logical(bk, shift) & 255
            plsc.addupdate_scatter(hist_v, [digit + lane_base], ones16,
                                   mask=active)
            return 0

        lax.fori_loop(0, nv_act, r_body, 0)
        reduce_and_publish()
        d, R = walk(R)
        prefix = lax.shift_left(prefix, 8) | d

    T = prefix
    tsw = T ^ MIN32

    def ex_body(i, carry):
        ngt, neq = carry
        bk = ckeys_v[pl.ds(i * 16, 16)]
        idxv = cidx_v[pl.ds(i * 16, 16)]
        valid = (i * 16 + lane) < nact
        m_gt = valid & ((bk ^ MIN32) > tsw)
        m_eq = valid & (bk == T)
        gi = m_gt.astype(jnp.int32)
        ei = m_eq.astype(jnp.int32)
        pos_g = ngt + plsc.cumsum(gi) - gi
        pos_e = neq + plsc.cumsum(ei) - ei
        plsc.store_scatter(gtv_v, [pos_g], _f32_from_bkey(bk), mask=m_gt)
        plsc.store_scatter(gti_v, [pos_g], idxv, mask=m_gt)
        plsc.store_scatter(eqi_v, [pos_e], idxv, mask=m_eq)
        return (ngt + jnp.sum(gi), neq + jnp.sum(ei))

    ngt, neq = lax.fori_loop(0, nv_act, ex_body,
                             (jnp.int32(0), jnp.int32(0)))

    cnts = (jnp.where(lane == 0, ngt, 0) + jnp.where(lane == 1, neq, 0))
    cntbuf_v[pl.ds(0, 16)] = cnts
    pltpu.sync_copy(cntbuf_v.at[pl.ds(0, 16)], scnt_s.at[pl.ds(tile * 16, 16)])
    plsc.subcore_barrier()

    pltpu.sync_copy(scnt_s, cntm_v)
    gt_off = jnp.int32(0)
    eq_pre = jnp.int32(0)
    total_gt = jnp.int32(0)
    for t in range(16):
        row = cntm_v[pl.ds(t * 16, 16)]
        g = row[0]
        e = row[1]
        before = jnp.int32(t) < tile
        gt_off = gt_off + jnp.where(before, g, 0)
        eq_pre = eq_pre + jnp.where(before, e, 0)
        total_gt = total_gt + g
    quota = jnp.clip(R - eq_pre, 0, neq)
    eq_off = total_gt + jnp.minimum(eq_pre, R)
    dump = jnp.int32(NCAND) + lane

    tf_v[pl.ds(0, 16)] = jnp.full((16,), _f32_from_bkey(T), jnp.float32)
    for j in range(8):
        k = j * 16 + lane

        @pl.when(j * 16 < ngt)
        def _(j=j, k=k):
            pos = jnp.where(k < ngt, jnp.clip(gt_off + k, 0, NOUT - 1), dump)
            pltpu.sync_copy(gtv_v.at[pl.ds(j * 16, 16)], soutv_s.at[pos])
            pltpu.sync_copy(gti_v.at[pl.ds(j * 16, 16)], souti_s.at[pos])

    for j in range(8):
        k = j * 16 + lane

        @pl.when(j * 16 < quota)
        def _(j=j, k=k):
            pos = jnp.where(k < quota, jnp.clip(eq_off + k, 0, NOUT - 1), dump)
            pltpu.sync_copy(eqi_v.at[pl.ds(j * 16, 16)], souti_s.at[pos])
            pltpu.sync_copy(tf_v, soutv_s.at[pos])

    plsc.subcore_barrier()

    @pl.when((tile == 0) & (cid == 0))
    def _writeout():
        pltpu.sync_copy(soutv_s.at[pl.ds(0, NCAND)], vals_hbm)
        pltpu.sync_copy(souti_s.at[pl.ds(0, NCAND)], idx_hbm)


_select = pl.kernel(
    _select_body,
    out_type=[
        jax.ShapeDtypeStruct((NCAND,), jnp.float32),
        jax.ShapeDtypeStruct((NCAND,), jnp.int32),
    ],
    mesh=plsc.VectorSubcoreMesh(core_axis_name="c", subcore_axis_name="s"),
    compiler_params=pltpu.CompilerParams(needs_layout_passes=False),
    scratch_types=[
        pltpu.VMEM((CHUNK,), jnp.float32),
        pltpu.VMEM((CHUNK,), jnp.int32),
        pltpu.VMEM((4096,), jnp.int32),
        pltpu.VMEM((256,), jnp.int32),
        pltpu.VMEM((4096,), jnp.int32),
        pltpu.VMEM((CHUNK + 16,), jnp.int32),
        pltpu.VMEM((CHUNK + 16,), jnp.int32),
        pltpu.VMEM((CHUNK + 16,), jnp.float32),
        pltpu.VMEM((CHUNK + 16,), jnp.int32),
        pltpu.VMEM((CHUNK + 16,), jnp.int32),
        pltpu.VMEM((256,), jnp.int32),
        pltpu.VMEM((16,), jnp.float32),
        pltpu.VMEM_SHARED((4096,), jnp.int32),
        pltpu.VMEM_SHARED((256,), jnp.int32),
        pltpu.VMEM_SHARED((NOUT,), jnp.float32),
        pltpu.VMEM_SHARED((NOUT,), jnp.int32),
    ],
)


def _epilogue_body(s_r_ref, i_r_ref, s_c_ref, i_c_ref, bbox_ref, mult_ref,
                   scale_ref, out_ref, lab_ref):
    s_r = s_r_ref[...]
    i_r = i_r_ref[...]
    s_c = s_c_ref[...]
    i_c = i_c_ref[...]

    beats = (s_c > s_r) | ((s_c == s_r) & (i_c < i_r))
    rank = jnp.sum(beats.astype(jnp.int32), axis=0, keepdims=True)

    row_iota = jax.lax.broadcasted_iota(jnp.int32, (NCAND, NCAND), 0)
    P = row_iota == rank
    sorted_s = jnp.sum(jnp.where(P, s_r, 0.0), axis=1, keepdims=True)
    sorted_i = jnp.sum(jnp.where(P, i_r, 0), axis=1, keepdims=True)

    top_s = sorted_s[:MAXK, :]
    top_i = sorted_i[:MAXK, :]
    bbox_idx = top_i // NUM_CLASSES
    label = top_i % NUM_CLASSES

    col_iota = jax.lax.broadcasted_iota(jnp.int32, (MAXK, NUM_QUERY), 1)
    G = (bbox_idx == col_iota).astype(jnp.float32)
    cxcywh = jax.lax.dot_general(
        G, bbox_ref[...], (((1,), (0,)), ((), ())),
        precision=jax.lax.Precision.HIGHEST)

    cxcy = cxcywh[:, 0:2]
    wh = cxcywh[:, 2:4]
    x1y1 = cxcy - wh / 2
    x2y2 = cxcy + wh / 2
    boxes = jnp.concatenate([x1y1, x2y2], axis=1)
    mult = mult_ref[...]
    boxes = boxes * mult
    boxes = jnp.clip(boxes, jnp.zeros_like(mult), mult)
    boxes = boxes / scale_ref[...]
    out_ref[...] = jnp.concatenate([boxes, top_s], axis=1)
    lab_ref[...] = label


def _epilogue(s128, i128, bbox, mult, scale):
    return pl.pallas_call(
        _epilogue_body,
        out_shape=[
            jax.ShapeDtypeStruct((MAXK, 5), jnp.float32),
            jax.ShapeDtypeStruct((MAXK, 1), jnp.int32),
        ],
    )(s128.reshape(1, NCAND), i128.reshape(1, NCAND),
      s128.reshape(NCAND, 1), i128.reshape(NCAND, 1),
      bbox, mult.reshape(1, 4), scale.reshape(1, 4))


def kernel(cls_logits, bbox_preds, scale, h, w):
    flat = cls_logits[0].reshape(-1)
    flat_pad = jnp.pad(flat, (0, NPAD - N),
                       constant_values=jnp.float32(-jnp.inf))
    vals, idx = _select(flat_pad)
    s128 = jax.nn.sigmoid(vals)
    i128 = idx

    wf = jnp.float32(w)
    hf = jnp.float32(h)
    mult = (jnp.array([1.0, 0.0, 1.0, 0.0], jnp.float32) * wf
            + jnp.array([0.0, 1.0, 0.0, 1.0], jnp.float32) * hf)

    bboxes, labels = _epilogue(s128, i128, bbox_preds[0], mult, scale)
    return bboxes, labels.reshape(MAXK)


# speedup vs baseline: 3.0010x; 1.0129x over previous
import functools

import jax
import jax.numpy as jnp
from jax import lax
from jax.experimental import pallas as pl
from jax.experimental.pallas import tpu as pltpu
from jax.experimental.pallas import tpu_sc as plsc

MAXK = 100
NCAND = 128
NUM_CLASSES = 80
NUM_QUERY = 900

N = NUM_QUERY * NUM_CLASSES
NT = 16
CHUNK = 4512
NV = CHUNK // 16
NPAD = NT * CHUNK
NOUT = 144
MIN32 = -(2 ** 31)
M31 = 0x7FFFFFFF


def _bkey_from_f32(x):
    b = lax.bitcast_convert_type(x, jnp.int32)
    mask = ((lax.shift_right_arithmetic(b, 31) & M31) | MIN32)
    return b ^ mask


def _f32_from_bkey(bk):
    sk = bk ^ MIN32
    i = jnp.where(sk >= 0, sk, sk ^ M31)
    return lax.bitcast_convert_type(i, jnp.float32)


def _select_body(x_hbm, vals_hbm, idx_hbm,
                 chunk_v, keys_v, hist_v, cntbuf_v, merge_v,
                 ckeys_v, cidx_v, gtv_v, gti_v, eqi_v, cntm_v, tf_v,
                 shist_s, scnt_s, soutv_s, souti_s):
    tile = lax.axis_index("s")
    cid = lax.axis_index("c")

    @pl.when(cid == 0)
    def _core0_body():
        _select_core0(x_hbm, vals_hbm, idx_hbm,
                      chunk_v, keys_v, hist_v, cntbuf_v, merge_v,
                      ckeys_v, cidx_v, gtv_v, gti_v, eqi_v, cntm_v, tf_v,
                      shist_s, scnt_s, soutv_s, souti_s, tile)


def _select_core0(x_hbm, vals_hbm, idx_hbm,
                  chunk_v, keys_v, hist_v, cntbuf_v, merge_v,
                  ckeys_v, cidx_v, gtv_v, gti_v, eqi_v, cntm_v, tf_v,
                  shist_s, scnt_s, soutv_s, souti_s, tile):
    base = tile * CHUNK
    lane = jnp.arange(16, dtype=jnp.int32)
    lane_base = lane * 256
    ones16 = jnp.ones((16,), jnp.int32)
    zeros16 = jnp.zeros((16,), jnp.int32)
    true16 = jnp.ones((16,), jnp.bool_)

    last_n = N - (NT - 1) * CHUNK
    nv_tile = jnp.where(tile == NT - 1, last_n // 16, NV)

    @pl.when(tile < NT - 1)
    def _load_full():
        pltpu.sync_copy(x_hbm.at[pl.ds(base, CHUNK)], chunk_v)

    @pl.when(tile == NT - 1)
    def _load_last():
        pltpu.sync_copy(x_hbm.at[pl.ds(base, last_n)],
                        chunk_v.at[pl.ds(0, last_n)])

    for w in range(256):
        hist_v[pl.ds(w * 16, 16)] = zeros16

    def r0_body(i, _):
        x = chunk_v[pl.ds(i * 16, 16)]
        bk = _bkey_from_f32(x)
        keys_v[pl.ds(i * 16, 16)] = bk
        digit = lax.shift_right_logical(bk, 24)
        plsc.addupdate_scatter(hist_v, [digit + lane_base], ones16,
                               mask=true16)
        return 0

    lax.fori_loop(0, nv_tile, r0_body, 0)

    def reduce_and_publish():
        for v in range(16):
            acc = zeros16
            for l in range(16):
                acc = acc + hist_v[pl.ds(l * 256 + v * 16, 16)]
            cntbuf_v[pl.ds(v * 16, 16)] = acc
        pltpu.sync_copy(cntbuf_v, shist_s.at[pl.ds(tile * 256, 256)])
        plsc.subcore_barrier()
        pltpu.sync_copy(shist_s, merge_v)
        plsc.subcore_barrier()

    def walk(R):
        blocks = []
        tots = []
        for v in range(16):
            c = merge_v[pl.ds(v * 16, 16)]
            for t in range(1, 16):
                c = c + merge_v[pl.ds(t * 256 + v * 16, 16)]
            blocks.append(c)
            tots.append(jnp.sum(c))
        suffix = [jnp.int32(0)] * 17
        for v in range(15, -1, -1):
            suffix[v] = suffix[v + 1] + tots[v]
        dstar = jnp.int32(0)
        gstar = jnp.int32(0)
        done = jnp.int32(0)
        for v in range(16):
            c = blocks[v]
            sfx = lax.rev(plsc.cumsum(lax.rev(c, (0,))), (0,)) - c
            gs = suffix[v + 1] + sfx
            m = gs < R
            mi = m.astype(jnp.int32)
            found = jnp.max(mi)
            first = jnp.sum(jnp.int32(1) - mi)
            gfirst = jnp.max(jnp.where(m, gs, jnp.int32(-1)))
            sel = found * (jnp.int32(1) - done)
            dstar = dstar + sel * (v * 16 + first)
            gstar = gstar + sel * gfirst
            done = jnp.maximum(done, found)
        return dstar, R - gstar

    reduce_and_publish()
    R = jnp.int32(NCAND)
    d1, R = walk(R)
    prefix = d1

    def compact_body(i, nact):
        bk = keys_v[pl.ds(i * 16, 16)]
        m = lax.shift_right_logical(bk, 24) >= d1
        mi = m.astype(jnp.int32)
        pos = nact + plsc.cumsum(mi) - mi
        idxv = base + i * 16 + lane
        plsc.store_scatter(ckeys_v, [pos], bk, mask=m)
        plsc.store_scatter(cidx_v, [pos], idxv, mask=m)
        return nact + jnp.sum(mi)

    nact = lax.fori_loop(0, nv_tile, compact_body, jnp.int32(0))
    nv_act = (nact + 15) // 16

    for rnd in range(1, 4):
        shift = 24 - 8 * rnd
        for w in range(256):
            hist_v[pl.ds(w * 16, 16)] = zeros16

        def r_body(i, _, shift=shift, prefix=prefix, nact=nact):
            bk = ckeys_v[pl.ds(i * 16, 16)]
            valid = (i * 16 + lane) < nact
            active = valid & (lax.shift_right_logical(bk, shift + 8) == prefix)
            digit = lax.shift_right_logical(bk, shift) & 255
            plsc.addupdate_scatter(hist_v, [digit + lane_base], ones16,
                                   mask=active)
            return 0

        lax.fori_loop(0, nv_act, r_body, 0)
        reduce_and_publish()
        d, R = walk(R)
        prefix = lax.shift_left(prefix, 8) | d

    T = prefix
    tsw = T ^ MIN32

    def ex_body(i, carry):
        ngt, neq = carry
        bk = ckeys_v[pl.ds(i * 16, 16)]
        idxv = cidx_v[pl.ds(i * 16, 16)]
        valid = (i * 16 + lane) < nact
        m_gt = valid & ((bk ^ MIN32) > tsw)
        m_eq = valid & (bk == T)
        gi = m_gt.astype(jnp.int32)
        ei = m_eq.astype(jnp.int32)
        pos_g = ngt + plsc.cumsum(gi) - gi
        pos_e = neq + plsc.cumsum(ei) - ei
        plsc.store_scatter(gtv_v, [pos_g], _f32_from_bkey(bk), mask=m_gt)
        plsc.store_scatter(gti_v, [pos_g], idxv, mask=m_gt)
        plsc.store_scatter(eqi_v, [pos_e], idxv, mask=m_eq)
        return (ngt + jnp.sum(gi), neq + jnp.sum(ei))

    ngt, neq = lax.fori_loop(0, nv_act, ex_body,
                             (jnp.int32(0), jnp.int32(0)))

    cnts = (jnp.where(lane == 0, ngt, 0) + jnp.where(lane == 1, neq, 0))
    cntbuf_v[pl.ds(0, 16)] = cnts
    pltpu.sync_copy(cntbuf_v.at[pl.ds(0, 16)], scnt_s.at[pl.ds(tile * 16, 16)])
    plsc.subcore_barrier()

    pltpu.sync_copy(scnt_s, cntm_v)
    gt_off = jnp.int32(0)
    eq_pre = jnp.int32(0)
    total_gt = jnp.int32(0)
    for t in range(16):
        row = cntm_v[pl.ds(t * 16, 16)]
        g = row[0]
        e = row[1]
        before = jnp.int32(t) < tile
        gt_off = gt_off + jnp.where(before, g, 0)
        eq_pre = eq_pre + jnp.where(before, e, 0)
        total_gt = total_gt + g
    quota = jnp.clip(R - eq_pre, 0, neq)
    eq_off = total_gt + jnp.minimum(eq_pre, R)
    dump = jnp.int32(NCAND) + lane

    tf_v[pl.ds(0, 16)] = jnp.full((16,), _f32_from_bkey(T), jnp.float32)
    for j in range(8):
        k = j * 16 + lane

        @pl.when(j * 16 < ngt)
        def _(j=j, k=k):
            pos = jnp.where(k < ngt, jnp.clip(gt_off + k, 0, NOUT - 1), dump)
            pltpu.sync_copy(gtv_v.at[pl.ds(j * 16, 16)], soutv_s.at[pos])
            pltpu.sync_copy(gti_v.at[pl.ds(j * 16, 16)], souti_s.at[pos])

    for j in range(8):
        k = j * 16 + lane

        @pl.when(j * 16 < quota)
        def _(j=j, k=k):
            pos = jnp.where(k < quota, jnp.clip(eq_off + k, 0, NOUT - 1), dump)
            pltpu.sync_copy(eqi_v.at[pl.ds(j * 16, 16)], souti_s.at[pos])
            pltpu.sync_copy(tf_v, soutv_s.at[pos])

    plsc.subcore_barrier()

    @pl.when(tile == 0)
    def _writeout():
        pltpu.sync_copy(soutv_s.at[pl.ds(0, NCAND)], vals_hbm)
        pltpu.sync_copy(souti_s.at[pl.ds(0, NCAND)], idx_hbm)


_select = pl.kernel(
    _select_body,
    out_type=[
        jax.ShapeDtypeStruct((NCAND,), jnp.float32),
        jax.ShapeDtypeStruct((NCAND,), jnp.int32),
    ],
    mesh=plsc.VectorSubcoreMesh(core_axis_name="c", subcore_axis_name="s"),
    compiler_params=pltpu.CompilerParams(needs_layout_passes=False),
    scratch_types=[
        pltpu.VMEM((CHUNK,), jnp.float32),
        pltpu.VMEM((CHUNK,), jnp.int32),
        pltpu.VMEM((4096,), jnp.int32),
        pltpu.VMEM((256,), jnp.int32),
        pltpu.VMEM((4096,), jnp.int32),
        pltpu.VMEM((CHUNK + 16,), jnp.int32),
        pltpu.VMEM((CHUNK + 16,), jnp.int32),
        pltpu.VMEM((CHUNK + 16,), jnp.float32),
        pltpu.VMEM((CHUNK + 16,), jnp.int32),
        pltpu.VMEM((CHUNK + 16,), jnp.int32),
        pltpu.VMEM((256,), jnp.int32),
        pltpu.VMEM((16,), jnp.float32),
        pltpu.VMEM_SHARED((4096,), jnp.int32),
        pltpu.VMEM_SHARED((256,), jnp.int32),
        pltpu.VMEM_SHARED((NOUT,), jnp.float32),
        pltpu.VMEM_SHARED((NOUT,), jnp.int32),
    ],
)


def _epilogue_body(s_r_ref, i_r_ref, s_c_ref, i_c_ref, bbox_ref, mult_ref,
                   scale_ref, out_ref, lab_ref):
    s_r = s_r_ref[...]
    i_r = i_r_ref[...]
    s_c = s_c_ref[...]
    i_c = i_c_ref[...]

    beats = (s_c > s_r) | ((s_c == s_r) & (i_c < i_r))
    rank = jnp.sum(beats.astype(jnp.int32), axis=0, keepdims=True)

    row_iota = jax.lax.broadcasted_iota(jnp.int32, (NCAND, NCAND), 0)
    P = row_iota == rank
    sorted_s = jnp.sum(jnp.where(P, s_r, 0.0), axis=1, keepdims=True)
    sorted_i = jnp.sum(jnp.where(P, i_r, 0), axis=1, keepdims=True)

    top_s = sorted_s[:MAXK, :]
    top_i = sorted_i[:MAXK, :]
    bbox_idx = top_i // NUM_CLASSES
    label = top_i % NUM_CLASSES

    col_iota = jax.lax.broadcasted_iota(jnp.int32, (MAXK, NUM_QUERY), 1)
    G = (bbox_idx == col_iota).astype(jnp.float32)
    cxcywh = jax.lax.dot_general(
        G, bbox_ref[...], (((1,), (0,)), ((), ())),
        precision=jax.lax.Precision.HIGHEST)

    cxcy = cxcywh[:, 0:2]
    wh = cxcywh[:, 2:4]
    x1y1 = cxcy - wh / 2
    x2y2 = cxcy + wh / 2
    boxes = jnp.concatenate([x1y1, x2y2], axis=1)
    mult = mult_ref[...]
    boxes = boxes * mult
    boxes = jnp.clip(boxes, jnp.zeros_like(mult), mult)
    boxes = boxes / scale_ref[...]
    out_ref[...] = jnp.concatenate([boxes, top_s], axis=1)
    lab_ref[...] = label


def _epilogue(s128, i128, bbox, mult, scale):
    return pl.pallas_call(
        _epilogue_body,
        out_shape=[
            jax.ShapeDtypeStruct((MAXK, 5), jnp.float32),
            jax.ShapeDtypeStruct((MAXK, 1), jnp.int32),
        ],
    )(s128.reshape(1, NCAND), i128.reshape(1, NCAND),
      s128.reshape(NCAND, 1), i128.reshape(NCAND, 1),
      bbox, mult.reshape(1, 4), scale.reshape(1, 4))


def kernel(cls_logits, bbox_preds, scale, h, w):
    flat = cls_logits[0].reshape(-1)
    vals, idx = _select(flat)
    s128 = jax.nn.sigmoid(vals)
    i128 = idx

    wf = jnp.float32(w)
    hf = jnp.float32(h)
    mult = (jnp.array([1.0, 0.0, 1.0, 0.0], jnp.float32) * wf
            + jnp.array([0.0, 1.0, 0.0, 1.0], jnp.float32) * hf)

    bboxes, labels = _epilogue(s128, i128, bbox_preds[0], mult, scale)
    return bboxes, labels.reshape(MAXK)
